# baseline (speedup 1.0000x reference)
import functools

import jax
import jax.numpy as jnp
from jax.experimental import pallas as pl
from jax.experimental.pallas import tpu as pltpu

ALPHA = 0.2


def _matmul_body(x_ref, w_ref, o_ref):
    o_ref[...] = jnp.dot(x_ref[...], w_ref[...],
                         preferred_element_type=jnp.float32)


def _tc_matmul(x, w, block_n=1000):
    n, k = x.shape
    k2, f = w.shape
    assert k == k2
    grid = (n // block_n,)
    return pl.pallas_call(
        _matmul_body,
        grid=grid,
        in_specs=[
            pl.BlockSpec((block_n, k), lambda i: (i, 0)),
            pl.BlockSpec((k, f), lambda i: (0, 0)),
        ],
        out_specs=pl.BlockSpec((block_n, f), lambda i: (i, 0)),
        out_shape=jax.ShapeDtypeStruct((n, f), jnp.float32),
    )(x, w)


def _gat_edges(h, src, dst, a, n):
    f = h.shape[1]
    s = h @ a[0, :f]
    d = h @ a[0, f:]
    logits = s[src] + d[dst]
    e = jnp.exp(-jax.nn.leaky_relu(logits, negative_slope=ALPHA))
    e_rowsum = jax.ops.segment_sum(e, src, num_segments=n)
    hp = jax.ops.segment_sum(e[:, None] * h[dst], src, num_segments=n)
    return hp / (e_rowsum[:, None] + 1e-16)


def kernel(x, adj, W0, a0, W1, a1, W2, a2, W3, a3, W_out, a_out, mlp_w, mlp_b):
    src, dst = adj[0], adj[1]
    n = x.shape[0]
    Wcat = jnp.concatenate([W0, W1, W2, W3], axis=1)
    H = _tc_matmul(x, Wcat)
    heads = []
    for i, a in enumerate((a0, a1, a2, a3)):
        h = H[:, 64 * i:64 * (i + 1)]
        heads.append(jax.nn.elu(_gat_edges(h, src, dst, a, n)))
    xcat = jnp.concatenate(heads, axis=1)
    H2 = _tc_matmul(xcat, W_out)
    x2 = jax.nn.elu(_gat_edges(H2, src, dst, a_out, n))
    return _tc_matmul(x2, mlp_w.T) + mlp_b
